# baseline (speedup 1.0000x reference)
import jax
import jax.numpy as jnp
from jax.experimental import pallas as pl
from jax.experimental.pallas import tpu as pltpu

_B, _D = 4096, 128
_CS = (2048, 1536, 512)
_CO = (0, 2048, 3584)
_C = 3
_MARGIN = 0.3
_EPS = 1e-6


def _triplet_kernel(a_hbm, p_hbm, n_hbm, out_ref, av, pv, nv, sem):
    for c in range(_C):
        sl = pl.ds(_CO[c], _CS[c])
        pltpu.make_async_copy(a_hbm.at[sl, :], av.at[sl, :], sem.at[0, c]).start()
        pltpu.make_async_copy(p_hbm.at[sl, :], pv.at[sl, :], sem.at[1, c]).start()
        pltpu.make_async_copy(n_hbm.at[sl, :], nv.at[sl, :], sem.at[2, c]).start()

    ones = jnp.ones((1, _D), jnp.float32)
    dims = (((1,), (1,)), ((), ()))
    ps = jnp.float32(0.0)
    nds = []
    for c in range(_C):
        sl = pl.ds(_CO[c], _CS[c])
        pltpu.make_async_copy(a_hbm.at[sl, :], av.at[sl, :], sem.at[0, c]).wait()
        pltpu.make_async_copy(p_hbm.at[sl, :], pv.at[sl, :], sem.at[1, c]).wait()
        a = av[sl, :]
        dp = a - pv[sl, :] + _EPS
        pd2 = jax.lax.dot_general(ones, dp * dp, dims,
                                  preferred_element_type=jnp.float32)
        ps = ps + jnp.sum(jnp.sqrt(pd2))
        pltpu.make_async_copy(n_hbm.at[sl, :], nv.at[sl, :], sem.at[2, c]).wait()
        dn = a - nv[sl, :] + _EPS
        nd2 = jax.lax.dot_general(ones, dn * dn, dims,
                                  preferred_element_type=jnp.float32)
        nds.append(jnp.sqrt(nd2))

    ndall = jnp.concatenate(nds, axis=1)
    inf = jnp.float32(jnp.inf)
    m1 = jnp.min(ndall)
    c1 = jnp.sum((ndall == m1).astype(jnp.float32))
    masked1 = jnp.where(ndall <= m1, inf, ndall)
    m2 = jnp.min(masked1)
    c2 = jnp.sum((masked1 == m2).astype(jnp.float32))
    masked2 = jnp.where(masked1 <= m2, inf, masked1)
    m3 = jnp.min(masked2)
    t1 = jnp.minimum(c1, 3.0)
    t2 = jnp.minimum(c2, 3.0 - t1)
    t3 = jnp.maximum(3.0 - t1 - t2, 0.0)
    m2s = jnp.where(t2 > 0.0, m2, 0.0)
    m3s = jnp.where(t3 > 0.0, m3, 0.0)
    neg_mean = (m1 * t1 + m2s * t2 + m3s * t3) * (1.0 / 3.0)
    pos_mean = ps * (1.0 / _B)
    loss = jnp.maximum(pos_mean - neg_mean + _MARGIN, 0.0)
    out_ref[...] = loss.reshape(1, 1)


@jax.jit
def kernel(anchor, positive, negative):
    out = pl.pallas_call(
        _triplet_kernel,
        in_specs=[pl.BlockSpec(memory_space=pltpu.HBM)] * 3,
        out_shape=jax.ShapeDtypeStruct((1, 1), jnp.float32),
        scratch_shapes=[
            pltpu.VMEM((_B, _D), jnp.float32),
            pltpu.VMEM((_B, _D), jnp.float32),
            pltpu.VMEM((_B, _D), jnp.float32),
            pltpu.SemaphoreType.DMA((3, 3)),
        ],
    )(anchor, positive, negative)
    return out[0, 0]


# speedup vs baseline: 1.0939x; 1.0939x over previous
import jax
import jax.numpy as jnp
from jax.experimental import pallas as pl
from jax.experimental.pallas import tpu as pltpu

_B, _D = 4096, 128
_BLK = 2048
_GRID = _B // _BLK
_MARGIN = 0.3
_EPS = 1e-6


def _triplet_kernel(a_ref, p_ref, n_ref, out_ref, nd_ref, acc_ref):
    i = pl.program_id(0)
    a = a_ref[:]
    dp = a - p_ref[:] + _EPS
    dn = a - n_ref[:] + _EPS
    ones = jnp.ones((1, _D), jnp.float32)
    dims = (((1,), (1,)), ((), ()))
    pd2 = jax.lax.dot_general(ones, dp * dp, dims,
                              preferred_element_type=jnp.float32)
    nd2 = jax.lax.dot_general(ones, dn * dn, dims,
                              preferred_element_type=jnp.float32)
    pd = jnp.sqrt(pd2)
    nd = jnp.sqrt(nd2)
    ps = jnp.sum(pd)

    @pl.when(i == 0)
    def _():
        acc_ref[0, 0] = ps

    @pl.when(i > 0)
    def _():
        acc_ref[0, 0] += ps

    nd_ref[pl.ds(i, 1), :] = nd

    @pl.when(i == _GRID - 1)
    def _():
        ndall = nd_ref[:]
        inf = jnp.float32(jnp.inf)
        m1 = jnp.min(ndall)
        c1 = jnp.sum((ndall == m1).astype(jnp.float32))
        masked1 = jnp.where(ndall <= m1, inf, ndall)
        m2 = jnp.min(masked1)
        c2 = jnp.sum((masked1 == m2).astype(jnp.float32))
        masked2 = jnp.where(masked1 <= m2, inf, masked1)
        m3 = jnp.min(masked2)
        t1 = jnp.minimum(c1, 3.0)
        t2 = jnp.minimum(c2, 3.0 - t1)
        t3 = jnp.maximum(3.0 - t1 - t2, 0.0)
        m2s = jnp.where(t2 > 0.0, m2, 0.0)
        m3s = jnp.where(t3 > 0.0, m3, 0.0)
        neg_mean = (m1 * t1 + m2s * t2 + m3s * t3) * (1.0 / 3.0)
        pos_mean = acc_ref[0, 0] * (1.0 / _B)
        loss = jnp.maximum(pos_mean - neg_mean + _MARGIN, 0.0)
        out_ref[...] = loss.reshape(1, 1)


@jax.jit
def kernel(anchor, positive, negative):
    out = pl.pallas_call(
        _triplet_kernel,
        grid=(_GRID,),
        in_specs=[pl.BlockSpec((_BLK, _D), lambda i: (i, 0))] * 3,
        out_specs=pl.BlockSpec((1, 1), lambda i: (0, 0)),
        out_shape=jax.ShapeDtypeStruct((1, 1), jnp.float32),
        scratch_shapes=[
            pltpu.VMEM((_GRID, _BLK), jnp.float32),
            pltpu.SMEM((1, 1), jnp.float32),
        ],
    )(anchor, positive, negative)
    return out[0, 0]


# speedup vs baseline: 1.0993x; 1.0049x over previous
import jax
import jax.numpy as jnp
from jax.experimental import pallas as pl
from jax.experimental.pallas import tpu as pltpu

_B, _D = 4096, 128
_BLK = 2048
_GRID = _B // _BLK
_MARGIN = 0.3
_EPS = 1e-6


def _triplet_kernel(a_ref, p_ref, n_ref, out_ref, nd_ref, acc_ref):
    i = pl.program_id(0)
    a = a_ref[:]
    dp = a - p_ref[:] + _EPS
    dn = a - n_ref[:] + _EPS
    ones = jnp.ones((1, _D), jnp.float32)
    dims = (((1,), (1,)), ((), ()))
    pd2 = jax.lax.dot_general(ones, dp * dp, dims,
                              preferred_element_type=jnp.float32)
    nd2 = jax.lax.dot_general(ones, dn * dn, dims,
                              preferred_element_type=jnp.float32)
    pd = jnp.sqrt(jnp.maximum(pd2, 0.0))
    nd = jnp.sqrt(jnp.maximum(nd2, 0.0))
    ps = jnp.sum(pd)

    @pl.when(i == 0)
    def _():
        acc_ref[0, 0] = ps

    @pl.when(i > 0)
    def _():
        acc_ref[0, 0] += ps

    nd_ref[pl.ds(i, 1), :] = nd

    @pl.when(i == _GRID - 1)
    def _():
        ndall = nd_ref[:]
        inf = jnp.float32(jnp.inf)
        m1 = jnp.min(ndall)
        c1 = jnp.sum((ndall == m1).astype(jnp.float32))
        masked1 = jnp.where(ndall <= m1, inf, ndall)
        m2 = jnp.min(masked1)
        c2 = jnp.sum((masked1 == m2).astype(jnp.float32))
        masked2 = jnp.where(masked1 <= m2, inf, masked1)
        m3 = jnp.min(masked2)
        t1 = jnp.minimum(c1, 3.0)
        t2 = jnp.minimum(c2, 3.0 - t1)
        t3 = jnp.maximum(3.0 - t1 - t2, 0.0)
        m2s = jnp.where(t2 > 0.0, m2, 0.0)
        m3s = jnp.where(t3 > 0.0, m3, 0.0)
        neg_mean = (m1 * t1 + m2s * t2 + m3s * t3) * (1.0 / 3.0)
        pos_mean = acc_ref[0, 0] * (1.0 / _B)
        loss = jnp.maximum(pos_mean - neg_mean + _MARGIN, 0.0)
        out_ref[...] = loss.reshape(1, 1)


@jax.jit
def kernel(anchor, positive, negative):
    out = pl.pallas_call(
        _triplet_kernel,
        grid=(_GRID,),
        in_specs=[pl.BlockSpec((_BLK, _D), lambda i: (i, 0))] * 3,
        out_specs=pl.BlockSpec((1, 1), lambda i: (0, 0)),
        out_shape=jax.ShapeDtypeStruct((1, 1), jnp.float32),
        scratch_shapes=[
            pltpu.VMEM((_GRID, _BLK), jnp.float32),
            pltpu.SMEM((1, 1), jnp.float32),
        ],
    )(anchor, positive, negative)
    return out[0, 0]
